# baseline (speedup 1.0000x reference)
import functools

import jax
import jax.numpy as jnp
from jax import lax
from jax.experimental import pallas as pl
from jax.experimental.pallas import tpu as pltpu
from jax.experimental.pallas import tpu_sc as plsc

VOCAB = 1000000
D = 64
B_ROWS = 4096
B_COLS = 200
TOTAL = B_ROWS * B_COLS
NW = 32
CHUNK = 128
PER_W = TOTAL // NW
NCHUNK = PER_W // CHUNK
NPAIR = NCHUNK // 2


def _embed_body(x_hbm, table_hbm, out_hbm, idx_v, rows0, rows1, gs0, gs1):
    wid = lax.axis_index("s") * 2 + lax.axis_index("c")
    pltpu.sync_copy(x_hbm.at[wid], idx_v)

    def gather(j, buf, sem):
        return pltpu.async_copy(table_hbm.at[idx_v.at[j]], buf, sem)

    gather(0, rows0, gs0)

    def pair(i, _):
        j0 = 2 * i
        j1 = j0 + 1
        gather(j1, rows1, gs1)
        pltpu.make_async_copy(table_hbm.at[idx_v.at[j0]], rows0, gs0).wait()
        pltpu.sync_copy(rows0, out_hbm.at[wid, j0])
        gather(jnp.minimum(j0 + 2, NCHUNK - 1), rows0, gs0)
        pltpu.make_async_copy(table_hbm.at[idx_v.at[j1]], rows1, gs1).wait()
        pltpu.sync_copy(rows1, out_hbm.at[wid, j1])
        return 0

    lax.fori_loop(0, NPAIR, pair, 0)
    pltpu.make_async_copy(table_hbm.at[idx_v.at[NCHUNK - 1]], rows0, gs0).wait()


@functools.partial(
    pl.kernel,
    out_type=jax.ShapeDtypeStruct((NW, NCHUNK, CHUNK, D), jnp.float32),
    mesh=plsc.VectorSubcoreMesh(core_axis_name="c", subcore_axis_name="s"),
    scratch_types=[
        pltpu.VMEM((NCHUNK, CHUNK), jnp.int32),
        pltpu.VMEM((CHUNK, D), jnp.float32),
        pltpu.VMEM((CHUNK, D), jnp.float32),
        pltpu.SemaphoreType.DMA,
        pltpu.SemaphoreType.DMA,
    ],
    compiler_params=pltpu.CompilerParams(use_tc_tiling_on_sc=False),
)
def _embed_sc(x_hbm, table_hbm, out_hbm, idx_v, rows0, rows1, gs0, gs1):
    _embed_body(x_hbm, table_hbm, out_hbm, idx_v, rows0, rows1, gs0, gs1)


def kernel(x, table):
    x_blocks = x.astype(jnp.int32).reshape(NW, NCHUNK, CHUNK)
    out = _embed_sc(x_blocks, table)
    return out.reshape(B_ROWS, B_COLS, D)


# speedup vs baseline: 1.0161x; 1.0161x over previous
import functools

import jax
import jax.numpy as jnp
from jax import lax
from jax.experimental import pallas as pl
from jax.experimental.pallas import tpu as pltpu
from jax.experimental.pallas import tpu_sc as plsc

VOCAB = 1000000
D = 64
B_ROWS = 4096
B_COLS = 200
TOTAL = B_ROWS * B_COLS
NW = 32
CHUNK = 128
PER_W = TOTAL // NW
NCHUNK = PER_W // CHUNK
NBUF = 8


def _embed_body(x_hbm, table_hbm, out_hbm, idx_v, rows, gsems, wsems):
    wid = lax.axis_index("s") * 2 + lax.axis_index("c")
    pltpu.sync_copy(x_hbm.at[wid], idx_v)

    def gather(j, b):
        pltpu.async_copy(table_hbm.at[idx_v.at[j]], rows[b], gsems[b])

    for b in range(NBUF):
        gather(b, b)

    def ring(i, _):
        j0 = NBUF * i
        for b in range(NBUF):
            j = j0 + b
            pltpu.make_async_copy(table_hbm.at[idx_v.at[j]], rows[b], gsems[b]).wait()
            pltpu.async_copy(rows[b], out_hbm.at[wid, j], wsems[b])
            pltpu.make_async_copy(rows[b], out_hbm.at[wid, j], wsems[b]).wait()
            gather(jnp.minimum(j + NBUF, NCHUNK - 1), b)
        return 0

    lax.fori_loop(0, NCHUNK // NBUF, ring, 0)
    for b in range(NBUF):
        pltpu.make_async_copy(
            table_hbm.at[idx_v.at[NCHUNK - 1]], rows[b], gsems[b]
        ).wait()


@functools.partial(
    pl.kernel,
    out_type=jax.ShapeDtypeStruct((NW, NCHUNK, CHUNK, D), jnp.float32),
    mesh=plsc.VectorSubcoreMesh(core_axis_name="c", subcore_axis_name="s"),
    scratch_types=(
        [pltpu.VMEM((NCHUNK, CHUNK), jnp.int32)]
        + [pltpu.VMEM((CHUNK, D), jnp.float32) for _ in range(NBUF)]
        + [pltpu.SemaphoreType.DMA for _ in range(2 * NBUF)]
    ),
    compiler_params=pltpu.CompilerParams(use_tc_tiling_on_sc=False),
)
def _embed_sc(x_hbm, table_hbm, out_hbm, idx_v, *scratch):
    rows = scratch[:NBUF]
    gsems = scratch[NBUF:2 * NBUF]
    wsems = scratch[2 * NBUF:]
    _embed_body(x_hbm, table_hbm, out_hbm, idx_v, rows, gsems, wsems)


def kernel(x, table):
    x_blocks = x.astype(jnp.int32).reshape(NW, NCHUNK, CHUNK)
    out = _embed_sc(x_blocks, table)
    return out.reshape(B_ROWS, B_COLS, D)


# speedup vs baseline: 1.0169x; 1.0007x over previous
import functools

import jax
import jax.numpy as jnp
from jax import lax
from jax.experimental import pallas as pl
from jax.experimental.pallas import tpu as pltpu
from jax.experimental.pallas import tpu_sc as plsc

VOCAB = 1000000
D = 64
B_ROWS = 4096
B_COLS = 200
TOTAL = B_ROWS * B_COLS
NW = 32
CHUNK = 512
PER_W = TOTAL // NW
NCHUNK = PER_W // CHUNK
NBUF = 2


def _embed_body(x_hbm, table_hbm, out_hbm, idx_v, rows, gsems, wsems):
    wid = lax.axis_index("s") * 2 + lax.axis_index("c")
    pltpu.sync_copy(x_hbm.at[wid], idx_v)

    def gather(j, b):
        pltpu.async_copy(table_hbm.at[idx_v.at[j]], rows[b], gsems[b])

    for b in range(NBUF):
        gather(b, b)

    def ring(i, _):
        j0 = NBUF * i
        for b in range(NBUF):
            j = j0 + b
            pltpu.make_async_copy(table_hbm.at[idx_v.at[j]], rows[b], gsems[b]).wait()
            pltpu.async_copy(rows[b], out_hbm.at[wid, j], wsems[b])
            pltpu.make_async_copy(rows[b], out_hbm.at[wid, j], wsems[b]).wait()
            gather(jnp.minimum(j + NBUF, NCHUNK - 1), b)
        return 0

    lax.fori_loop(0, NCHUNK // NBUF, ring, 0)
    for b in range(NBUF):
        pltpu.make_async_copy(
            table_hbm.at[idx_v.at[NCHUNK - 1]], rows[b], gsems[b]
        ).wait()


@functools.partial(
    pl.kernel,
    out_type=jax.ShapeDtypeStruct((NW, NCHUNK, CHUNK, D), jnp.float32),
    mesh=plsc.VectorSubcoreMesh(core_axis_name="c", subcore_axis_name="s"),
    scratch_types=(
        [pltpu.VMEM((NCHUNK, CHUNK), jnp.int32)]
        + [pltpu.VMEM((CHUNK, D), jnp.float32) for _ in range(NBUF)]
        + [pltpu.SemaphoreType.DMA for _ in range(2 * NBUF)]
    ),
    compiler_params=pltpu.CompilerParams(use_tc_tiling_on_sc=False),
)
def _embed_sc(x_hbm, table_hbm, out_hbm, idx_v, *scratch):
    rows = scratch[:NBUF]
    gsems = scratch[NBUF:2 * NBUF]
    wsems = scratch[2 * NBUF:]
    _embed_body(x_hbm, table_hbm, out_hbm, idx_v, rows, gsems, wsems)


def kernel(x, table):
    x_blocks = x.astype(jnp.int32).reshape(NW, NCHUNK, CHUNK)
    out = _embed_sc(x_blocks, table)
    return out.reshape(B_ROWS, B_COLS, D)
